# baseline (speedup 1.0000x reference)
import functools

import jax
import jax.numpy as jnp
from jax import lax
from jax.experimental import pallas as pl
from jax.experimental.pallas import tpu as pltpu
from jax.experimental.pallas import tpu_sc as plsc

jax.config.update("jax_enable_x64", True)

ROWS, COLS = 16384, 100
N = ROWS * COLS
NC, NS, L = 2, 16, 16
NW = NC * NS
CHUNK = N // NW
NVEC = CHUNK // L
UNROLL = 4

M = 999999
FOLD = 48577
LIMB_C = [pow(2, 8 * k, M) for k in range(8)]


def _hash16(v):
    x = v
    digits = []
    r = x
    for d in (100000, 10000, 1000, 100, 10):
        q = (r.astype(jnp.float32) * jnp.float32(1.0 / d)).astype(jnp.int32)
        rr = r - q * d
        c = rr >= d
        q = jnp.where(c, q + 1, q)
        r = jnp.where(c, rr - d, rr)
        digits.append(q)
    digits.append(r)

    lo = jnp.full((L,), 0x84222325, dtype=jnp.uint32)
    hi = jnp.full((L,), 0xCBF29CE4, dtype=jnp.uint32)
    for idx, p in enumerate(range(5, -1, -1)):
        byte = digits[idx].astype(jnp.uint32) + 48
        lo2 = lo ^ byte
        nlo = lo2 * 435
        t1 = (lo2 >> 16) * 435
        t0 = (lo2 & 0xFFFF) * 435
        carry = (t1 + (t0 >> 16)) >> 16
        nhi = hi * 435 + carry + (lo2 << 8)
        if p > 0:
            valid = x >= (10 ** p)
            lo = jnp.where(valid, nlo, lo)
            hi = jnp.where(valid, nhi, hi)
        else:
            lo, hi = nlo, nhi

    s = lo & 0xFF
    s = s + ((lo >> 8) & 0xFF) * LIMB_C[1]
    s = s + ((lo >> 16) & 0xFF) * LIMB_C[2]
    s = s + (lo >> 24) * LIMB_C[3]
    s = s + (hi & 0xFF) * LIMB_C[4]
    s = s + ((hi >> 8) & 0xFF) * LIMB_C[5]
    s = s + ((hi >> 16) & 0xFF) * LIMB_C[6]
    s = s + (hi >> 24) * LIMB_C[7]
    for _ in range(3):
        s = (s >> 20) * FOLD + (s & 0xFFFFF)
    s = jnp.where(s >= M, s - M, s)
    return jnp.where(x != 0, s.astype(jnp.int32) + 1, 0)


PIECE_E = CHUNK // 2
PIECE_W = 2 * PIECE_E
NVEC_P = PIECE_E // L


@functools.partial(
    pl.kernel,
    out_type=jax.ShapeDtypeStruct((2 * N,), jnp.int32),
    mesh=plsc.VectorSubcoreMesh(core_axis_name="c", subcore_axis_name="s"),
    scratch_types=[
        pltpu.VMEM((PIECE_W,), jnp.int32),
        pltpu.VMEM((PIECE_W,), jnp.int32),
    ],
    compiler_params=pltpu.CompilerParams(needs_layout_passes=False),
)
def _sc_hash(x_hbm, out_hbm, in_v, out_v):
    wid = lax.axis_index("s") * jnp.int32(NC) + lax.axis_index("c")
    base = wid * jnp.int32(2 * CHUNK)
    iota2 = lax.iota(jnp.int32, L) * 2
    zero = jnp.zeros((L,), jnp.int32)
    for p in range(2):
        off = base + jnp.int32(p * PIECE_W)
        pltpu.sync_copy(x_hbm.at[pl.ds(off, PIECE_W)], in_v)

        def body(i, carry):
            for j in range(UNROLL):
                vb = i * jnp.int32(UNROLL * 2 * L) + jnp.int32(j * 2 * L)
                idx = vb + iota2
                val = plsc.load_gather(in_v, [idx])
                res = _hash16(val)
                plsc.store_scatter(out_v, [idx], res)
                plsc.store_scatter(out_v, [idx + 1], zero)
            return carry

        lax.fori_loop(jnp.int32(0), jnp.int32(NVEC_P // UNROLL), body, jnp.int32(0))
        pltpu.sync_copy(out_v, out_hbm.at[pl.ds(off, PIECE_W)])


def kernel(x):
    xb = lax.bitcast_convert_type(x, jnp.int32).reshape(2 * N)
    out = _sc_hash(xb)
    return lax.bitcast_convert_type(out.reshape(ROWS, COLS, 2), jnp.int64)


# speedup vs baseline: 7.3753x; 7.3753x over previous
import functools

import jax
import jax.numpy as jnp
from jax import lax
from jax.experimental import pallas as pl
from jax.experimental.pallas import tpu as pltpu
from jax.experimental.pallas import tpu_sc as plsc

jax.config.update("jax_enable_x64", True)

ROWS, COLS = 16384, 100
N = ROWS * COLS
NC, NS, L = 2, 16, 16
NW = NC * NS
CHUNK = N // NW
NVEC = CHUNK // L
UNROLL = 8

M = 999999
FOLD = 48577
LIMB_C = [pow(2, 8 * k, M) for k in range(8)]


def _hash16(v):
    x = v
    digits = []
    r = x
    for d in (100000, 10000, 1000, 100, 10):
        q = (r.astype(jnp.float32) * jnp.float32(1.0 / d)).astype(jnp.int32)
        r = r - q * d
        digits.append(q)
    digits.append(r)

    lo = jnp.full((L,), 0x84222325, dtype=jnp.uint32)
    hi = jnp.full((L,), 0xCBF29CE4, dtype=jnp.uint32)
    for idx, p in enumerate(range(5, -1, -1)):
        byte = digits[idx].astype(jnp.uint32) + 48
        lo2 = lo ^ byte
        nlo = lo2 * 435
        t1 = (lo2 >> 16) * 435
        t0 = nlo - (t1 << 16)
        carry = (t1 + (t0 >> 16)) >> 16
        nhi = hi * 435 + carry + (lo2 << 8)
        if p > 0:
            valid = x >= (10 ** p)
            lo = jnp.where(valid, nlo, lo)
            hi = jnp.where(valid, nhi, hi)
        else:
            lo, hi = nlo, nhi

    s = lo & 0xFF
    s = s + ((lo >> 8) & 0xFF) * LIMB_C[1]
    s = s + ((lo >> 16) & 0xFF) * LIMB_C[2]
    s = s + (lo >> 24) * LIMB_C[3]
    s = s + (hi & 0xFF) * LIMB_C[4]
    s = s + ((hi >> 8) & 0xFF) * LIMB_C[5]
    s = s + ((hi >> 16) & 0xFF) * LIMB_C[6]
    s = s + (hi >> 24) * LIMB_C[7]
    for _ in range(3):
        s = (s >> 20) * FOLD + (s & 0xFFFFF)
    s = jnp.where(s >= M, s - M, s)
    return jnp.where(x != 0, s.astype(jnp.int32) + 1, 0)


@functools.partial(
    pl.kernel,
    out_type=jax.ShapeDtypeStruct((N,), jnp.int32),
    mesh=plsc.VectorSubcoreMesh(core_axis_name="c", subcore_axis_name="s"),
    scratch_types=[
        pltpu.VMEM((CHUNK,), jnp.int32),
        pltpu.VMEM((CHUNK,), jnp.int32),
    ],
)
def _sc_hash(x_hbm, out_hbm, in_v, out_v):
    wid = lax.axis_index("s") * jnp.int32(NC) + lax.axis_index("c")
    base = wid * jnp.int32(CHUNK)
    pltpu.sync_copy(x_hbm.at[pl.ds(base, CHUNK)], in_v)

    def body(i, carry):
        for j in range(UNROLL):
            off = i * jnp.int32(UNROLL * L) + jnp.int32(j * L)
            out_v[pl.ds(off, L)] = _hash16(in_v[pl.ds(off, L)])
        return carry

    lax.fori_loop(jnp.int32(0), jnp.int32(NVEC // UNROLL), body, jnp.int32(0))
    pltpu.sync_copy(out_v, out_hbm.at[pl.ds(base, CHUNK)])


def kernel(x):
    x32 = x.astype(jnp.int32).reshape(N)
    out = _sc_hash(x32)
    return out.reshape(ROWS, COLS).astype(jnp.int64)


# speedup vs baseline: 9.0171x; 1.2226x over previous
import functools

import jax
import jax.numpy as jnp
from jax import lax
from jax.experimental import pallas as pl
from jax.experimental.pallas import tpu as pltpu
from jax.experimental.pallas import tpu_sc as plsc

jax.config.update("jax_enable_x64", True)

ROWS, COLS = 16384, 100
N = ROWS * COLS
NC, NS, L = 2, 16, 16
NW = NC * NS
CHUNK = N // NW
NVEC = CHUNK // L
UNROLL = 8

M = 999999
FOLD = 48577
LIMB_C = [pow(2, 8 * k, M) for k in range(8)]


def _hash16(v):
    x = v
    digits = []
    r = x
    for d in (100000, 10000, 1000, 100, 10):
        q = (r.astype(jnp.float32) * jnp.float32(1.0 / d)).astype(jnp.int32)
        r = r - q * d
        digits.append(q)
    digits.append(r)

    lo = jnp.full((L,), 0x84222325, dtype=jnp.uint32)
    hi = jnp.full((L,), 0xCBF29CE4, dtype=jnp.uint32)
    for idx, p in enumerate(range(5, -1, -1)):
        byte = digits[idx].astype(jnp.uint32) + 48
        lo2 = lo ^ byte
        nlo = lo2 * 435
        t1 = (lo2 >> 16) * 435
        t0 = nlo - (t1 << 16)
        carry = (t1 + (t0 >> 16)) >> 16
        nhi = hi * 435 + carry + (lo2 << 8)
        if p > 0:
            valid = x >= (10 ** p)
            lo = jnp.where(valid, nlo, lo)
            hi = jnp.where(valid, nhi, hi)
        else:
            lo, hi = nlo, nhi

    s = lo & 0xFF
    s = s + ((lo >> 8) & 0xFF) * LIMB_C[1]
    s = s + ((lo >> 16) & 0xFF) * LIMB_C[2]
    s = s + (lo >> 24) * LIMB_C[3]
    s = s + (hi & 0xFF) * LIMB_C[4]
    s = s + ((hi >> 8) & 0xFF) * LIMB_C[5]
    s = s + ((hi >> 16) & 0xFF) * LIMB_C[6]
    s = s + (hi >> 24) * LIMB_C[7]
    for _ in range(3):
        s = (s >> 20) * FOLD + (s & 0xFFFFF)
    s = jnp.where(s >= M, s - M, s)
    return jnp.where(x != 0, s.astype(jnp.int32) + 1, 0)


@functools.partial(
    pl.kernel,
    out_type=jax.ShapeDtypeStruct((N,), jnp.int32),
    mesh=plsc.VectorSubcoreMesh(core_axis_name="c", subcore_axis_name="s"),
    scratch_types=[
        pltpu.VMEM((CHUNK,), jnp.int32),
        pltpu.VMEM((CHUNK,), jnp.int32),
    ],
)
def _sc_hash(x_hbm, out_hbm, in_v, out_v):
    wid = lax.axis_index("s") * jnp.int32(NC) + lax.axis_index("c")
    base = wid * jnp.int32(CHUNK)
    pltpu.sync_copy(x_hbm.at[pl.ds(base, CHUNK)], in_v)

    def body(i, carry):
        for j in range(UNROLL):
            off = i * jnp.int32(UNROLL * L) + jnp.int32(j * L)
            out_v[pl.ds(off, L)] = _hash16(in_v[pl.ds(off, L)])
        return carry

    lax.fori_loop(jnp.int32(0), jnp.int32(NVEC // UNROLL), body, jnp.int32(0))
    pltpu.sync_copy(out_v, out_hbm.at[pl.ds(base, CHUNK)])


def kernel(x):
    xt = x.T
    x32 = xt.astype(jnp.int32).reshape(N)
    out = _sc_hash(x32)
    return out.reshape(COLS, ROWS).astype(jnp.int64).T
